# Initial kernel scaffold; baseline (speedup 1.0000x reference)
import jax
import jax.numpy as jnp
from jax.experimental import pallas as pl


def kernel(boxes, scores):
    raise NotImplementedError("write your pallas kernel here")


# speedup vs baseline: 96.6950x; 96.6950x over previous
import jax
import jax.numpy as jnp
from jax.experimental import pallas as pl

_N = 5000
_B = 256
_NB = 20
_NPAD = _B * _NB
_IOU_TH = 0.4
_SCORE_TH = 0.3


def _iou_gt(rk, cm):
    x1a, y1a, x2a, y2a, aa = (rk[:, 0:1], rk[:, 1:2], rk[:, 2:3],
                              rk[:, 3:4], rk[:, 4:5])
    x1b, y1b, x2b, y2b, ab = (cm[0:1, :], cm[1:2, :], cm[2:3, :],
                              cm[3:4, :], cm[4:5, :])
    w = jnp.maximum(jnp.minimum(x2a, x2b) - jnp.maximum(x1a, x1b), 0.0)
    h = jnp.maximum(jnp.minimum(y2a, y2b) - jnp.maximum(y1a, y1b), 0.0)
    inter = w * h
    union = aa + ab - inter
    iou = inter / (union + 1e-6)
    return (iou > _IOU_TH).astype(jnp.float32)


def _nms_body(rows_ref, cols_ref, keep_ref):
    keep_ref[...] = jnp.ones((_NB, 1, _B), jnp.float32)
    ii = jax.lax.broadcasted_iota(jnp.int32, (_B, _B), 0)
    jj = jax.lax.broadcasted_iota(jnp.int32, (_B, _B), 1)
    tri = (ii < jj).astype(jnp.float32)

    def block_step(k, carry):
        rk = rows_ref[k]
        m_kk = _iou_gt(rk, cols_ref[k]) * tri
        k_init = keep_ref[k]

        def fp_cond(c):
            return c[1]

        def fp_body(c):
            kv, _ = c
            sup = jnp.dot(kv, m_kk, preferred_element_type=jnp.float32)
            kn = k_init * (sup == 0.0).astype(jnp.float32)
            return kn, jnp.sum(jnp.abs(kn - kv)) > 0.0

        kv, _ = jax.lax.while_loop(fp_cond, fp_body,
                                   (k_init, jnp.asarray(True)))
        keep_ref[k] = kv

        def later(m, inner):
            sup = jnp.dot(kv, _iou_gt(rk, cols_ref[m]),
                          preferred_element_type=jnp.float32)
            keep_ref[m] = keep_ref[m] * (sup == 0.0).astype(jnp.float32)
            return inner

        jax.lax.fori_loop(k + 1, _NB, later, 0)
        return carry

    jax.lax.fori_loop(0, _NB, block_step, 0)

    def thresh(k, carry):
        keep_ref[k] = keep_ref[k] * (
            cols_ref[k][5:6, :] > _SCORE_TH).astype(jnp.float32)
        return carry

    jax.lax.fori_loop(0, _NB, thresh, 0)


def kernel(boxes, scores):
    order = jnp.argsort(-scores)
    b = jnp.take(boxes, order, axis=0)
    s = jnp.take(scores, order, axis=0)
    pad = _NPAD - _N
    bp = jnp.concatenate([b, jnp.zeros((pad, 4), jnp.float32)], axis=0)
    sp = jnp.concatenate([s, jnp.zeros((pad,), jnp.float32)], axis=0)
    area = (bp[:, 2] - bp[:, 0]) * (bp[:, 3] - bp[:, 1])
    feat = jnp.concatenate(
        [bp, area[:, None], sp[:, None], jnp.zeros((_NPAD, 2), jnp.float32)],
        axis=1)
    rows = feat.reshape(_NB, _B, 8)
    cols = jnp.transpose(rows, (0, 2, 1))
    keep = pl.pallas_call(
        _nms_body,
        out_shape=jax.ShapeDtypeStruct((_NB, 1, _B), jnp.float32),
    )(rows, cols)
    kf = keep.reshape(_NPAD)[:_N]
    return jnp.concatenate([b * kf[:, None], (s * kf)[:, None]], axis=1)
